# baseline (speedup 1.0000x reference)
import functools

import jax
import jax.numpy as jnp
from jax import lax
from jax.experimental import pallas as pl
from jax.experimental.pallas import tpu as pltpu
from jax.experimental.pallas import tpu_sc as plsc

_HIDDEN = 128
_N_NODES = 10000
_N_EDGES = 320000

_NC = 2
_NS = 16
_L = 16
_NW = _NC * _NS
_EPW = _N_EDGES // _NW
_UNROLL = 5


def _proj_body(xf_ref, xt_ref, w_ref, b_ref, p_ref):
    wf = w_ref[:, :_HIDDEN]
    wt = w_ref[:, _HIDDEN:]
    dn = (((1,), (1,)), ((), ()))
    p_ref[:, :_N_NODES] = (
        lax.dot_general(wf, xf_ref[...], dn, preferred_element_type=jnp.float32)
        + b_ref[0, 0]
    )
    p_ref[:, _N_NODES:] = lax.dot_general(
        wt, xt_ref[...], dn, preferred_element_type=jnp.float32
    )


_project = pl.pallas_call(
    _proj_body,
    out_shape=jax.ShapeDtypeStruct((1, 2 * _N_NODES), jnp.float32),
    in_specs=[
        pl.BlockSpec(memory_space=pltpu.VMEM),
        pl.BlockSpec(memory_space=pltpu.VMEM),
        pl.BlockSpec(memory_space=pltpu.VMEM),
        pl.BlockSpec(memory_space=pltpu.SMEM),
    ],
    out_specs=pl.BlockSpec(memory_space=pltpu.VMEM),
)

_mesh = plsc.VectorSubcoreMesh(
    core_axis_name="c", subcore_axis_name="s", num_cores=_NC, num_subcores=_NS
)


@functools.partial(
    pl.kernel,
    mesh=_mesh,
    compiler_params=pltpu.CompilerParams(needs_layout_passes=False),
    out_type=jax.ShapeDtypeStruct((_N_EDGES,), jnp.float32),
    scratch_types=[
        pltpu.VMEM((2 * _N_NODES,), jnp.float32),
        pltpu.VMEM((_EPW,), jnp.int32),
        pltpu.VMEM((_EPW,), jnp.int32),
        pltpu.VMEM((_EPW,), jnp.float32),
        pltpu.SemaphoreType.DMA,
    ],
)
def _edge_gather(p_hbm, idx_hbm, out_hbm, p_v, i0_v, i1_v, out_v, sem):
    wid = lax.axis_index("s") * _NC + lax.axis_index("c")
    base = wid * _EPW
    cp_p = pltpu.async_copy(p_hbm, p_v, sem)
    cp_i0 = pltpu.async_copy(idx_hbm.at[pl.ds(base, _EPW)], i0_v, sem)
    cp_i1 = pltpu.async_copy(idx_hbm.at[pl.ds(_N_EDGES + base, _EPW)], i1_v, sem)
    cp_p.wait()
    cp_i0.wait()
    cp_i1.wait()

    @plsc.parallel_loop(0, _EPW, _L * _UNROLL, unroll=2)
    def _gather_loop(i):
        for u in range(_UNROLL):
            sl = pl.ds(i + u * _L, _L)
            a = plsc.load_gather(p_v, [i0_v[sl]])
            c = plsc.load_gather(p_v, [i1_v[sl]])
            out_v[sl] = a + c
    pltpu.sync_copy(out_v, out_hbm.at[pl.ds(base, _EPW)])


def kernel(x_from, x_to, edge_label_index, W, b):
    p = _project(x_from, x_to, W, b.reshape(1, 1))
    idx = edge_label_index.astype(jnp.int32)
    idx_flat = jnp.concatenate([idx[0], idx[1] + _N_NODES])
    return _edge_gather(p.reshape(2 * _N_NODES), idx_flat)


# speedup vs baseline: 1.1330x; 1.1330x over previous
import functools

import jax
import jax.numpy as jnp
from jax import lax
from jax.experimental import pallas as pl
from jax.experimental.pallas import tpu as pltpu
from jax.experimental.pallas import tpu_sc as plsc

_HIDDEN = 128
_N_NODES = 10000
_N_EDGES = 320000

_NC = 2
_NS = 16
_L = 16
_NW = _NC * _NS
_EPW = _N_EDGES // _NW
_UNROLL = 5
_NGRP = 79


def _proj_body(xf_ref, xt_ref, w_ref, b_ref, p_ref):
    wf = w_ref[:, :_HIDDEN]
    wt = w_ref[:, _HIDDEN:]
    dn = (((1,), (1,)), ((), ()))
    p_ref[:, :_N_NODES] = (
        lax.dot_general(wf, xf_ref[...], dn, preferred_element_type=jnp.float32)
        + b_ref[0, 0]
    )
    p_ref[:, _N_NODES:] = lax.dot_general(
        wt, xt_ref[...], dn, preferred_element_type=jnp.float32
    )


_project = pl.pallas_call(
    _proj_body,
    out_shape=jax.ShapeDtypeStruct((1, 2 * _N_NODES), jnp.float32),
    in_specs=[
        pl.BlockSpec(memory_space=pltpu.VMEM),
        pl.BlockSpec(memory_space=pltpu.VMEM),
        pl.BlockSpec(memory_space=pltpu.VMEM),
        pl.BlockSpec(memory_space=pltpu.SMEM),
    ],
    out_specs=pl.BlockSpec(memory_space=pltpu.VMEM),
)

_mesh = plsc.VectorSubcoreMesh(
    core_axis_name="c", subcore_axis_name="s", num_cores=_NC, num_subcores=_NS
)


@functools.partial(
    pl.kernel,
    mesh=_mesh,
    compiler_params=pltpu.CompilerParams(needs_layout_passes=False),
    out_type=jax.ShapeDtypeStruct((_N_EDGES,), jnp.float32),
    scratch_types=[
        pltpu.VMEM((2 * _N_NODES,), jnp.float32),
        pltpu.VMEM((_NGRP * 256,), jnp.int32),
        pltpu.VMEM((_EPW,), jnp.float32),
        pltpu.SemaphoreType.DMA,
    ],
)
def _edge_gather(p_hbm, idx_hbm, out_hbm, p_v, slab_v, out_v, sem):
    wid = lax.axis_index("s") * _NC + lax.axis_index("c")
    base = wid * _EPW
    g0 = base // 128
    e_off = base - g0 * 128
    cp_p = pltpu.async_copy(p_hbm.at[0], p_v, sem)
    cp_i = pltpu.async_copy(idx_hbm.at[pl.ds(g0 * 256, _NGRP * 256)], slab_v, sem)
    cp_p.wait()
    cp_i.wait()

    @plsc.parallel_loop(0, _EPW, _L * _UNROLL, unroll=2)
    def _gather_loop(j):
        for u in range(_UNROLL):
            pos = e_off + j + u * _L
            addr = pos + (pos // 128) * 128
            a = plsc.load_gather(p_v, [slab_v[pl.ds(addr, _L)]])
            c = plsc.load_gather(p_v, [slab_v[pl.ds(addr + 128, _L)] + _N_NODES])
            out_v[pl.ds(j + u * _L, _L)] = a + c
    pltpu.sync_copy(out_v, out_hbm.at[pl.ds(base, _EPW)])


def kernel(x_from, x_to, edge_label_index, W, b):
    p = _project(x_from, x_to, W, b.reshape(1, 1))
    idx = edge_label_index.astype(jnp.int32)
    idx_flat = jnp.transpose(
        idx.reshape(2, _N_EDGES // 128, 128), (1, 0, 2)
    ).reshape(2 * _N_EDGES)
    return _edge_gather(p, idx_flat)


# speedup vs baseline: 1.6353x; 1.4433x over previous
import functools

import jax
import jax.numpy as jnp
from jax import lax
from jax.experimental import pallas as pl
from jax.experimental.pallas import tpu as pltpu
from jax.experimental.pallas import tpu_sc as plsc

_HIDDEN = 128
_N_NODES = 10000
_N_EDGES = 320000

_NC = 2
_NS = 16
_L = 16
_NW = _NC * _NS
_EPW = _N_EDGES // _NW
_UNROLL = 5
_NGRP = 79


def _proj_body(xf_ref, xt_ref, w_ref, b_ref, p_ref):
    wf = w_ref[:, :_HIDDEN]
    wt = w_ref[:, _HIDDEN:]
    dn = (((1,), (1,)), ((), ()))
    p_ref[:, :_N_NODES] = (
        lax.dot_general(wf, xf_ref[...], dn, preferred_element_type=jnp.float32)
        + b_ref[0, 0]
    )
    p_ref[:, _N_NODES:] = lax.dot_general(
        wt, xt_ref[...], dn, preferred_element_type=jnp.float32
    )


_project = pl.pallas_call(
    _proj_body,
    out_shape=jax.ShapeDtypeStruct((1, 2 * _N_NODES), jnp.float32),
    in_specs=[
        pl.BlockSpec(memory_space=pltpu.VMEM),
        pl.BlockSpec(memory_space=pltpu.VMEM),
        pl.BlockSpec(memory_space=pltpu.VMEM),
        pl.BlockSpec(memory_space=pltpu.SMEM),
    ],
    out_specs=pl.BlockSpec(memory_space=pltpu.VMEM),
)

_mesh = plsc.VectorSubcoreMesh(
    core_axis_name="c", subcore_axis_name="s", num_cores=_NC, num_subcores=_NS
)


@functools.partial(
    pl.kernel,
    mesh=_mesh,
    compiler_params=pltpu.CompilerParams(needs_layout_passes=False),
    out_type=jax.ShapeDtypeStruct((_N_EDGES,), jnp.float32),
    scratch_types=[
        pltpu.VMEM((2 * _N_NODES,), jnp.float32),
        pltpu.VMEM((_NGRP, 2, 128), jnp.int32),
        pltpu.VMEM((_EPW,), jnp.float32),
        pltpu.SemaphoreType.DMA,
    ],
)
def _edge_gather(p_hbm, idx_hbm, out_hbm, p_v, slab_v, out_v, sem):
    wid = lax.axis_index("s") * _NC + lax.axis_index("c")
    base = wid * _EPW
    g0 = lax.shift_right_logical(base, 7)
    e_off = lax.bitwise_and(base, 127)
    cp_p = pltpu.async_copy(p_hbm.at[0], p_v, sem)
    cp_i = pltpu.async_copy(idx_hbm.at[pl.ds(g0, _NGRP)], slab_v, sem)
    cp_p.wait()
    cp_i.wait()

    @plsc.parallel_loop(0, _EPW, _L * _UNROLL, unroll=2)
    def _gather_loop(j):
        for u in range(_UNROLL):
            pos = e_off + j + u * _L
            gl = lax.shift_right_logical(pos, 7)
            c0 = lax.bitwise_and(pos, 127)
            a = plsc.load_gather(p_v, [slab_v[gl, 0, pl.ds(c0, _L)]])
            c = plsc.load_gather(p_v, [slab_v[gl, 1, pl.ds(c0, _L)] + _N_NODES])
            out_v[pl.ds(j + u * _L, _L)] = a + c
    pltpu.sync_copy(out_v, out_hbm.at[pl.ds(base, _EPW)])


def kernel(x_from, x_to, edge_label_index, W, b):
    p = _project(x_from, x_to, W, b.reshape(1, 1))
    idx = edge_label_index.astype(jnp.int32)
    idx3 = jnp.transpose(idx.reshape(2, _N_EDGES // 128, 128), (1, 0, 2))
    return _edge_gather(p, idx3)


# speedup vs baseline: 1.6456x; 1.0063x over previous
import functools

import jax
import jax.numpy as jnp
from jax import lax
from jax.experimental import pallas as pl
from jax.experimental.pallas import tpu as pltpu
from jax.experimental.pallas import tpu_sc as plsc

_HIDDEN = 128
_N_NODES = 10000
_N_EDGES = 320000

_NC = 2
_NS = 16
_L = 16
_NW = _NC * _NS
_EPW = _N_EDGES // _NW
_UNROLL = 5
_NGRP = 79
_ECHUNK = 6000
_NGA = 48
_GB0 = 46
_NGB = 33


def _proj_body(xf_ref, xt_ref, w_ref, b_ref, p_ref):
    wf = w_ref[:, :_HIDDEN]
    wt = w_ref[:, _HIDDEN:]
    dn = (((1,), (1,)), ((), ()))
    p_ref[:, :_N_NODES] = (
        lax.dot_general(wf, xf_ref[...], dn, preferred_element_type=jnp.float32)
        + b_ref[0, 0]
    )
    p_ref[:, _N_NODES:] = lax.dot_general(
        wt, xt_ref[...], dn, preferred_element_type=jnp.float32
    )


_project = pl.pallas_call(
    _proj_body,
    out_shape=jax.ShapeDtypeStruct((1, 2 * _N_NODES), jnp.float32),
    in_specs=[
        pl.BlockSpec(memory_space=pltpu.VMEM),
        pl.BlockSpec(memory_space=pltpu.VMEM),
        pl.BlockSpec(memory_space=pltpu.VMEM),
        pl.BlockSpec(memory_space=pltpu.SMEM),
    ],
    out_specs=pl.BlockSpec(memory_space=pltpu.VMEM),
)

_mesh = plsc.VectorSubcoreMesh(
    core_axis_name="c", subcore_axis_name="s", num_cores=_NC, num_subcores=_NS
)


@functools.partial(
    pl.kernel,
    mesh=_mesh,
    compiler_params=pltpu.CompilerParams(needs_layout_passes=False),
    out_type=jax.ShapeDtypeStruct((_N_EDGES,), jnp.float32),
    scratch_types=[
        pltpu.VMEM((2 * _N_NODES,), jnp.float32),
        pltpu.VMEM((_NGA, 2, 128), jnp.int32),
        pltpu.VMEM((_NGB, 2, 128), jnp.int32),
        pltpu.VMEM((_EPW,), jnp.float32),
        pltpu.SemaphoreType.DMA,
        pltpu.SemaphoreType.DMA,
    ],
)
def _edge_gather(p_hbm, idx_hbm, out_hbm,
                 p_v, slab_a, slab_b, out_v, sem, sem_out):
    wid = lax.axis_index("s") * _NC + lax.axis_index("c")
    base = wid * _EPW
    g0 = lax.shift_right_logical(base, 7)
    e_off = lax.bitwise_and(base, 127)
    cp_p = pltpu.async_copy(p_hbm.at[0], p_v, sem)
    cp_a = pltpu.async_copy(idx_hbm.at[pl.ds(g0, _NGA)], slab_a, sem)
    cp_b = pltpu.async_copy(idx_hbm.at[pl.ds(g0 + _GB0, _NGB)], slab_b, sem)
    cp_p.wait()
    cp_a.wait()

    @plsc.parallel_loop(0, _ECHUNK, _L * _UNROLL, unroll=2)
    def _gather_a(j):
        for u in range(_UNROLL):
            pos = e_off + j + u * _L
            gl = lax.shift_right_logical(pos, 7)
            c0 = lax.bitwise_and(pos, 127)
            a = plsc.load_gather(p_v, [slab_a[gl, 0, pl.ds(c0, _L)]])
            c = plsc.load_gather(p_v, [slab_a[gl, 1, pl.ds(c0, _L)] + _N_NODES])
            out_v[pl.ds(j + u * _L, _L)] = a + c

    cp_out = pltpu.async_copy(
        out_v.at[pl.ds(0, _ECHUNK)], out_hbm.at[pl.ds(base, _ECHUNK)], sem_out
    )
    cp_b.wait()

    @plsc.parallel_loop(_ECHUNK, _EPW, _L * _UNROLL, unroll=2)
    def _gather_b(j):
        for u in range(_UNROLL):
            pos = e_off + j + u * _L
            gl = lax.shift_right_logical(pos, 7) - _GB0
            c0 = lax.bitwise_and(pos, 127)
            a = plsc.load_gather(p_v, [slab_b[gl, 0, pl.ds(c0, _L)]])
            c = plsc.load_gather(p_v, [slab_b[gl, 1, pl.ds(c0, _L)] + _N_NODES])
            out_v[pl.ds(j + u * _L, _L)] = a + c

    pltpu.sync_copy(
        out_v.at[pl.ds(_ECHUNK, _EPW - _ECHUNK)],
        out_hbm.at[pl.ds(base + _ECHUNK, _EPW - _ECHUNK)],
    )
    cp_out.wait()


def kernel(x_from, x_to, edge_label_index, W, b):
    p = _project(x_from, x_to, W, b.reshape(1, 1))
    idx = edge_label_index.astype(jnp.int32)
    idx3 = jnp.transpose(idx.reshape(2, _N_EDGES // 128, 128), (1, 0, 2))
    return _edge_gather(p, idx3)
